# baseline (speedup 1.0000x reference)
import functools

import jax
import jax.numpy as jnp
from jax import lax
from jax.experimental import pallas as pl
from jax.experimental.pallas import tpu as pltpu
from jax.experimental.pallas import tpu_sc as plsc

_LANES = 16
_NBUF = 2


@functools.lru_cache(maxsize=None)
def _make_pool_kernel(n_seg: int, seg_rows: int, units: int, chunk_rows: int):
    n_cores = 2
    cols = units // n_cores
    n_grp = cols // _LANES
    n_chunks = seg_rows // chunk_rows
    mesh = plsc.VectorSubcoreMesh(core_axis_name="c", subcore_axis_name="s")

    @functools.partial(
        pl.kernel,
        mesh=mesh,
        out_type=jax.ShapeDtypeStruct((n_seg, units), jnp.float32),
        scratch_types=[
            pltpu.VMEM((chunk_rows, cols), jnp.float32),
            pltpu.VMEM((chunk_rows, cols), jnp.float32),
            pltpu.VMEM((_LANES,), jnp.float32),
            pltpu.VMEM((cols,), jnp.float32),
            pltpu.SemaphoreType.DMA,
            pltpu.SemaphoreType.DMA,
        ],
    )
    def pool(x_hbm, sz_hbm, out_hbm, buf0, buf1, szv, outv, sem0, sem1):
        core = lax.axis_index("c")
        sub = lax.axis_index("s")
        seg = sub
        row0 = seg * seg_rows
        col0 = core * cols
        bufs = (buf0, buf1)
        sems = (sem0, sem1)

        def start(ci, b):
            src = x_hbm.at[pl.ds(row0 + ci * chunk_rows, chunk_rows),
                           pl.ds(col0, cols)]
            return pltpu.async_copy(src, bufs[b], sems[b])

        handles = [start(0, 0), start(1, 1)]
        acc = tuple(jnp.zeros((_LANES,), jnp.float32) for _ in range(n_grp))

        for ci in range(n_chunks):
            b = ci % _NBUF
            handles[b].wait()
            buf = bufs[b]

            def body(r, carry, buf=buf):
                return tuple(
                    carry[g] + buf[r, pl.ds(g * _LANES, _LANES)]
                    for g in range(n_grp)
                )

            acc = lax.fori_loop(0, chunk_rows, body, acc)
            if ci + _NBUF < n_chunks:
                handles[b] = start(ci + _NBUF, b)

        pltpu.sync_copy(sz_hbm.at[seg], szv)
        s = szv[...]
        for g in range(n_grp):
            outv[pl.ds(g * _LANES, _LANES)] = acc[g] / s
        pltpu.sync_copy(outv, out_hbm.at[seg, pl.ds(col0, cols)])

    return pool


def kernel(x, nclasses, nfeature):
    n_seg = nclasses.shape[0]
    units = x.shape[1]
    seg_rows = x.shape[0] // n_seg
    sizes = (nclasses * nfeature).astype(jnp.float32)
    sz_b = jnp.broadcast_to(sizes[:, None], (n_seg, _LANES))
    chunk_rows = 200
    pool = _make_pool_kernel(n_seg, seg_rows, units, chunk_rows)
    return pool(x, sz_b)
